# baseline (speedup 1.0000x reference)
import functools

import jax
import jax.numpy as jnp
from jax import lax
from jax.experimental import pallas as pl
from jax.experimental.pallas import tpu as pltpu
from jax.experimental.pallas import tpu_sc as plsc

N = 10000
E = 320000
D = 128
R = 8
K = N * R

NC, NS, L = 2, 16, 16
NW = NC * NS
EPW = E // NW
BLK = 80
CH = 2000
BPC = CH // BLK
NCH = EPW // CH
NP = 10240
RPS = NP // NS
ZR = RPS // 5
KPS = K // NS

_mesh = plsc.VectorSubcoreMesh(core_axis_name="c", subcore_axis_name="s",
                               num_cores=NC, num_subcores=NS)
_params = pltpu.CompilerParams(needs_layout_passes=False)


def _copy80(src_ref, src_off, dst_ref):
    for t in range(BLK // L):
        dst_ref[pl.ds(t * L, L)] = src_ref[pl.ds(src_off + t * L, L)]


@functools.partial(
    pl.kernel,
    out_type=jax.ShapeDtypeStruct((E,), jnp.float32),
    mesh=_mesh,
    compiler_params=_params,
    scratch_types=[
        pltpu.VMEM((K,), jnp.float32),
        pltpu.VMEM((CH,), jnp.int32),
        pltpu.VMEM((CH,), jnp.float32),
    ],
)
def _sc_scales(inv_hbm, k_hbm, out_hbm, inv_v, kst_v, s_v):
    c = lax.axis_index("c")
    s = lax.axis_index("s")
    wid = s * NC + c
    pltpu.sync_copy(inv_hbm, inv_v)

    def outer(ch, _):
        pltpu.sync_copy(k_hbm.at[pl.ds(wid * EPW + ch * CH, CH)], kst_v)

        def inner(q, _):
            idx = kst_v[pl.ds(q * L, L)]
            s_v[pl.ds(q * L, L)] = plsc.load_gather(inv_v, [idx])
            return 0
        lax.fori_loop(0, CH // L, inner, 0)
        pltpu.sync_copy(s_v, out_hbm.at[pl.ds(wid * EPW + ch * CH, CH)])
        return 0
    lax.fori_loop(0, NCH, outer, 0)


@functools.partial(
    pl.kernel,
    out_type=jax.ShapeDtypeStruct((NC * NP, D), jnp.float32),
    mesh=_mesh,
    compiler_params=_params,
    scratch_types=[
        pltpu.VMEM_SHARED((NP, D), jnp.float32),
        pltpu.VMEM((CH,), jnp.int32),
        pltpu.VMEM((CH,), jnp.int32),
        pltpu.VMEM((CH,), jnp.float32),
        pltpu.VMEM((BLK,), jnp.int32),
        pltpu.VMEM((BLK,), jnp.int32),
        pltpu.VMEM((BLK, D), jnp.float32),
        pltpu.VMEM((BLK, D), jnp.float32),
        pltpu.VMEM((ZR, D), jnp.float32),
        pltpu.SemaphoreType.DMA,
        pltpu.SemaphoreType.DMA,
        pltpu.SemaphoreType.DMA,
        pltpu.SemaphoreType.DMA,
    ],
)
def _sc_scatter(h_hbm, g_hbm, d_hbm, s_hbm, out_hbm,
                acc_sh, gst_v, dst_v, s_v, didx0_v, didx1_v,
                rows0_v, rows1_v, zero_v, semg0, semg1, sems0, sems1):
    c = lax.axis_index("c")
    s = lax.axis_index("s")
    wid = s * NC + c
    zeros = jnp.zeros((L,), jnp.float32)

    def fill_zero(i, _):
        for t in range(D // L):
            zero_v[i, pl.ds(t * L, L)] = zeros
        return 0
    lax.fori_loop(0, ZR, fill_zero, 0)

    def zero_acc(i, _):
        pltpu.sync_copy(zero_v, acc_sh.at[pl.ds(s * RPS + i * ZR, ZR)])
        return 0
    lax.fori_loop(0, 5, zero_acc, 0)
    plsc.subcore_barrier()

    bufs = ((didx0_v, rows0_v, semg0, sems0), (didx1_v, rows1_v, semg1, sems1))

    def outer(ch, _):
        base = wid * EPW + ch * CH
        pltpu.sync_copy(g_hbm.at[pl.ds(base, CH)], gst_v)
        pltpu.sync_copy(d_hbm.at[pl.ds(base, CH)], dst_v)
        pltpu.sync_copy(s_hbm.at[pl.ds(base, CH)], s_v)
        pltpu.async_copy(h_hbm.at[gst_v.at[pl.ds(0, BLK)]], rows0_v, semg0)

        def step(kk, b):
            didx_v, rows_v, semg, sems = bufs[b]
            o_didx_v, o_rows_v, o_semg, o_sems = bufs[1 - b]
            pltpu.make_async_copy(h_hbm.at[pl.ds(0, BLK)], rows_v,
                                  semg).wait()

            @pl.when(kk < BPC - 1)
            def _():
                @pl.when(kk >= 1)
                def _():
                    pltpu.make_async_copy(out_hbm.at[pl.ds(0, BLK)],
                                          o_rows_v, o_sems).wait()
                pltpu.async_copy(
                    h_hbm.at[gst_v.at[pl.ds((kk + 1) * BLK, BLK)]],
                    o_rows_v, o_semg)

            def scale(j, _):
                sv = plsc.load_gather(
                    s_v, [jnp.full((L,), kk * BLK + j, jnp.int32)])
                for t in range(D // L):
                    sl = pl.ds(t * L, L)
                    rows_v[j, sl] = rows_v[j, sl] * sv
                return 0
            lax.fori_loop(0, BLK, scale, 0)
            _copy80(dst_v, kk * BLK, didx_v)
            pltpu.async_copy(rows_v, acc_sh.at[didx_v], sems, add=True)

        def inner(kk, _):
            @pl.when(kk % 2 == 0)
            def _():
                step(kk, 0)

            @pl.when(kk % 2 == 1)
            def _():
                step(kk, 1)
            return 0
        lax.fori_loop(0, BPC, inner, 0)
        for b in range(2):
            didx_v, rows_v, semg, sems = bufs[b]
            pltpu.make_async_copy(out_hbm.at[pl.ds(0, BLK)], rows_v,
                                  sems).wait()
        return 0
    lax.fori_loop(0, NCH, outer, 0)
    plsc.subcore_barrier()

    def read_out(i, _):
        pltpu.sync_copy(acc_sh.at[pl.ds(s * RPS + i * ZR, ZR)], zero_v)
        pltpu.sync_copy(zero_v, out_hbm.at[pl.ds(c * NP + s * RPS + i * ZR,
                                                 ZR)])
        return 0
    lax.fori_loop(0, 5, read_out, 0)


def _tc_transform_body(x_ref, w_ref, b_ref, o_ref):
    r = pl.program_id(1)
    acc = jnp.dot(x_ref[...], w_ref[0], preferred_element_type=jnp.float32)
    sel = jnp.where(r == R, 1.0, 0.0).astype(jnp.float32)
    o_ref[...] = acc + sel * b_ref[0][None, :]


def _tc_transform(x, w_all, b):
    nb = 10
    blk = N // nb
    return pl.pallas_call(
        _tc_transform_body,
        grid=(nb, R + 1),
        in_specs=[
            pl.BlockSpec((blk, D), lambda i, r: (i, 0)),
            pl.BlockSpec((1, D, D), lambda i, r: (r, 0, 0)),
            pl.BlockSpec((1, D), lambda i, r: (0, 0)),
        ],
        out_specs=pl.BlockSpec((blk, D), lambda i, r: (r * nb + i, 0)),
        out_shape=jax.ShapeDtypeStruct(((R + 1) * N, D), jnp.float32),
    )(x, w_all, b.reshape(1, D))


def _tc_combine_body(a_ref, p0_ref, p1_ref, o_ref):
    o_ref[...] = jnp.maximum(a_ref[...] + p0_ref[...] + p1_ref[...], 0.0)


def _tc_combine(a, p0, p1):
    nb = 10
    blk = N // nb
    return pl.pallas_call(
        _tc_combine_body,
        grid=(nb,),
        in_specs=[pl.BlockSpec((blk, D), lambda i: (i, 0))] * 3,
        out_specs=pl.BlockSpec((blk, D), lambda i: (i, 0)),
        out_shape=jax.ShapeDtypeStruct((N, D), jnp.float32),
    )(a, p0, p1)


def kernel(x, edge_index, edge_type, W1_rel, W1_root, b1, W2_rel, W2_root, b2):
    src = edge_index[0].astype(jnp.int32)
    dst = edge_index[1].astype(jnp.int32)
    et = edge_type.astype(jnp.int32)
    g = et * N + src
    k = dst * R + et

    ohtab = jnp.zeros((L * R, D), jnp.float32
                      ).at[:R].set(jnp.repeat(jnp.eye(R, dtype=jnp.float32),
                                              L, axis=1))
    cnt_parts = _sc_scatter(ohtab, et, dst,
                            jnp.ones((E,), jnp.float32)).reshape(NC, NP, R, L)
    cnt = (cnt_parts[0, :N, :, 0] + cnt_parts[1, :N, :, 0]).reshape(K)
    inv = 1.0 / jnp.maximum(cnt, 1.0)
    s_all = _sc_scales(inv, k)

    w_all1 = jnp.concatenate([W1_rel, W1_root[None]], axis=0)
    w_all2 = jnp.concatenate([W2_rel, W2_root[None]], axis=0)

    h = x
    for w_all, b in ((w_all1, b1), (w_all2, b2)):
        hf = _tc_transform(h, w_all, b)
        parts = _sc_scatter(hf, g, dst, s_all)
        h = _tc_combine(hf[R * N:], parts[:N], parts[NP:NP + N])
    return h


# speedup vs baseline: 2.6716x; 2.6716x over previous
import functools

import jax
import jax.numpy as jnp
from jax import lax
from jax.experimental import pallas as pl
from jax.experimental.pallas import tpu as pltpu
from jax.experimental.pallas import tpu_sc as plsc

N = 10000
E = 320000
D = 128
R = 8
K = N * R

NC, NS, L = 2, 16, 16
NW = NC * NS
EPW = E // NW
BLK = 80
CH = 2000
BPC = CH // BLK
NCH = EPW // CH
NP = 10240
RPS = NP // NS
ZR = RPS // 5
KPS = K // NS

_mesh = plsc.VectorSubcoreMesh(core_axis_name="c", subcore_axis_name="s",
                               num_cores=NC, num_subcores=NS)
_params = pltpu.CompilerParams(needs_layout_passes=False)


def _copy80(src_ref, src_off, dst_ref):
    for t in range(BLK // L):
        dst_ref[pl.ds(t * L, L)] = src_ref[pl.ds(src_off + t * L, L)]


@functools.partial(
    pl.kernel,
    out_type=jax.ShapeDtypeStruct((E,), jnp.float32),
    mesh=_mesh,
    compiler_params=_params,
    scratch_types=[
        pltpu.VMEM((K,), jnp.float32),
        pltpu.VMEM((CH,), jnp.int32),
        pltpu.VMEM((CH,), jnp.float32),
    ],
)
def _sc_scales(inv_hbm, k_hbm, out_hbm, inv_v, kst_v, s_v):
    c = lax.axis_index("c")
    s = lax.axis_index("s")
    wid = s * NC + c
    pltpu.sync_copy(inv_hbm, inv_v)

    def outer(ch, _):
        pltpu.sync_copy(k_hbm.at[pl.ds(wid * EPW + ch * CH, CH)], kst_v)

        def inner(q, _):
            idx = kst_v[pl.ds(q * L, L)]
            s_v[pl.ds(q * L, L)] = plsc.load_gather(inv_v, [idx])
            return 0
        lax.fori_loop(0, CH // L, inner, 0)
        pltpu.sync_copy(s_v, out_hbm.at[pl.ds(wid * EPW + ch * CH, CH)])
        return 0
    lax.fori_loop(0, NCH, outer, 0)


@functools.partial(
    pl.kernel,
    out_type=jax.ShapeDtypeStruct((NC * NP, D), jnp.float32),
    mesh=_mesh,
    compiler_params=_params,
    scratch_types=[
        pltpu.VMEM_SHARED((NP, D), jnp.float32),
        pltpu.VMEM((CH,), jnp.int32),
        pltpu.VMEM((CH,), jnp.int32),
        pltpu.VMEM((CH,), jnp.float32),
        pltpu.VMEM((BLK,), jnp.int32),
        pltpu.VMEM((BLK,), jnp.int32),
        pltpu.VMEM((BLK, D), jnp.float32),
        pltpu.VMEM((BLK, D), jnp.float32),
        pltpu.VMEM((ZR, D), jnp.float32),
        pltpu.SemaphoreType.DMA,
        pltpu.SemaphoreType.DMA,
        pltpu.SemaphoreType.DMA,
        pltpu.SemaphoreType.DMA,
    ],
)
def _sc_scatter(h_hbm, g_hbm, d_hbm, s_hbm, out_hbm,
                acc_sh, gst_v, dst_v, s_v, didx0_v, didx1_v,
                rows0_v, rows1_v, zero_v, semg0, semg1, sems0, sems1):
    c = lax.axis_index("c")
    s = lax.axis_index("s")
    wid = s * NC + c
    zeros = jnp.zeros((L,), jnp.float32)

    def fill_zero(i, _):
        for t in range(D // L):
            zero_v[i, pl.ds(t * L, L)] = zeros
        return 0
    lax.fori_loop(0, ZR, fill_zero, 0)

    def zero_acc(i, _):
        pltpu.sync_copy(zero_v, acc_sh.at[pl.ds(s * RPS + i * ZR, ZR)])
        return 0
    lax.fori_loop(0, 5, zero_acc, 0)
    plsc.subcore_barrier()

    bufs = ((didx0_v, rows0_v, semg0, sems0), (didx1_v, rows1_v, semg1, sems1))

    def outer(ch, _):
        base = wid * EPW + ch * CH
        pltpu.sync_copy(g_hbm.at[pl.ds(base, CH)], gst_v)
        pltpu.sync_copy(d_hbm.at[pl.ds(base, CH)], dst_v)
        pltpu.sync_copy(s_hbm.at[pl.ds(base, CH)], s_v)
        pltpu.async_copy(h_hbm.at[gst_v.at[pl.ds(0, BLK)]], rows0_v, semg0)

        def step(kk, b):
            didx_v, rows_v, semg, sems = bufs[b]
            o_didx_v, o_rows_v, o_semg, o_sems = bufs[1 - b]
            pltpu.make_async_copy(h_hbm.at[pl.ds(0, BLK)], rows_v,
                                  semg).wait()

            @pl.when(kk < BPC - 1)
            def _():
                @pl.when(kk >= 1)
                def _():
                    pltpu.make_async_copy(out_hbm.at[pl.ds(0, BLK)],
                                          o_rows_v, o_sems).wait()
                pltpu.async_copy(
                    h_hbm.at[gst_v.at[pl.ds((kk + 1) * BLK, BLK)]],
                    o_rows_v, o_semg)

            def scale(j, _):
                sv = plsc.load_gather(
                    s_v, [jnp.full((L,), kk * BLK + j, jnp.int32)])
                for t in range(D // L):
                    sl = pl.ds(t * L, L)
                    rows_v[j, sl] = rows_v[j, sl] * sv
                return 0
            lax.fori_loop(0, BLK, scale, 0)
            _copy80(dst_v, kk * BLK, didx_v)
            pltpu.async_copy(rows_v, acc_sh.at[didx_v], sems, add=True)

        def inner(kk, _):
            @pl.when(kk % 2 == 0)
            def _():
                step(kk, 0)

            @pl.when(kk % 2 == 1)
            def _():
                step(kk, 1)
            return 0
        lax.fori_loop(0, BPC, inner, 0)
        for b in range(2):
            didx_v, rows_v, semg, sems = bufs[b]
            pltpu.make_async_copy(out_hbm.at[pl.ds(0, BLK)], rows_v,
                                  sems).wait()
        return 0
    lax.fori_loop(0, NCH, outer, 0)
    plsc.subcore_barrier()

    def read_out(i, _):
        pltpu.sync_copy(acc_sh.at[pl.ds(s * RPS + i * ZR, ZR)], zero_v)
        pltpu.sync_copy(zero_v, out_hbm.at[pl.ds(c * NP + s * RPS + i * ZR,
                                                 ZR)])
        return 0
    lax.fori_loop(0, 5, read_out, 0)


def _tc_transform_body(x_ref, w_ref, b_ref, o_ref):
    r = pl.program_id(1)
    acc = jnp.dot(x_ref[...], w_ref[0], preferred_element_type=jnp.float32)
    sel = jnp.where(r == R, 1.0, 0.0).astype(jnp.float32)
    o_ref[...] = acc + sel * b_ref[0][None, :]


def _tc_transform(x, w_all, b):
    nb = 10
    blk = N // nb
    return pl.pallas_call(
        _tc_transform_body,
        grid=(nb, R + 1),
        in_specs=[
            pl.BlockSpec((blk, D), lambda i, r: (i, 0)),
            pl.BlockSpec((1, D, D), lambda i, r: (r, 0, 0)),
            pl.BlockSpec((1, D), lambda i, r: (0, 0)),
        ],
        out_specs=pl.BlockSpec((blk, D), lambda i, r: (r * nb + i, 0)),
        out_shape=jax.ShapeDtypeStruct(((R + 1) * N, D), jnp.float32),
    )(x, w_all, b.reshape(1, D))


def _tc_combine_body(a_ref, p0_ref, p1_ref, o_ref):
    o_ref[...] = jnp.maximum(a_ref[...] + p0_ref[...] + p1_ref[...], 0.0)


def _tc_combine(a, p0, p1):
    nb = 10
    blk = N // nb
    return pl.pallas_call(
        _tc_combine_body,
        grid=(nb,),
        in_specs=[pl.BlockSpec((blk, D), lambda i: (i, 0))] * 3,
        out_specs=pl.BlockSpec((blk, D), lambda i: (i, 0)),
        out_shape=jax.ShapeDtypeStruct((N, D), jnp.float32),
    )(a, p0, p1)


def kernel(x, edge_index, edge_type, W1_rel, W1_root, b1, W2_rel, W2_root, b2):
    src = edge_index[0].astype(jnp.int32)
    dst = edge_index[1].astype(jnp.int32)
    et = edge_type.astype(jnp.int32)
    g = et * N + src
    k = dst * R + et

    REP = 1250
    ohtab = jnp.tile(jnp.repeat(jnp.eye(R, dtype=jnp.float32), L, axis=1),
                     (REP, 1))
    g_cnt = (jnp.arange(E, dtype=jnp.int32) % REP) * R + et
    cnt_parts = _sc_scatter(ohtab, g_cnt, dst,
                            jnp.ones((E,), jnp.float32)).reshape(NC, NP, R, L)
    cnt = (cnt_parts[0, :N, :, 0] + cnt_parts[1, :N, :, 0]).reshape(K)
    inv = 1.0 / jnp.maximum(cnt, 1.0)
    s_all = _sc_scales(inv, k)

    w_all1 = jnp.concatenate([W1_rel, W1_root[None]], axis=0)
    w_all2 = jnp.concatenate([W2_rel, W2_root[None]], axis=0)

    h = x
    for w_all, b in ((w_all1, b1), (w_all2, b2)):
        hf = _tc_transform(h, w_all, b)
        parts = _sc_scatter(hf, g, dst, s_all)
        h = _tc_combine(hf[R * N:], parts[:N], parts[NP:NP + N])
    return h
